# baseline (speedup 1.0000x reference)
import jax
import jax.numpy as jnp
from jax import lax
from jax.experimental import pallas as pl
from jax.experimental.pallas import tpu as pltpu
from jax.experimental.pallas import tpu_sc as plsc

N_FIELDS = 26
VOCAB = 100000
D = 32
B = 16384

ROWS = N_FIELDS * B
NW = 32
RPW = ROWS // NW
C = 256
H = C // 128
CPB = B // C
NCHUNK = RPW // C
NBUF = 2
NOUTER = NCHUNK // NBUF
DT = D // 8


def _sc_body(idx_hbm, eps_hbm, mean_hbm, lv_hbm,
             means_out, lvs_out, zs_out,
             idx_v, m_in, l_in, e_t, m_t, l_t, z_t,
             in_sem, out_sem):
    wid = lax.axis_index("s") * 2 + lax.axis_index("c")
    base = wid * RPW
    pltpu.sync_copy(idx_hbm.at[pl.ds(base, RPW)], idx_v)

    lane = lax.iota(jnp.int32, 16)

    def field_off(c):
        q = wid * NCHUNK + c
        f = q // CPB
        bt0 = (q % CPB) * H
        return f, bt0

    def start_in(c, b):
        f, bt0 = field_off(c)
        for h in range(H):
            isl = idx_v.at[pl.ds(c * C + h * 128, 128)]
            hs = pl.ds(h * 128, 128)
            pltpu.async_copy(mean_hbm.at[f].at[isl], m_in.at[b, hs], in_sem.at[b])
            pltpu.async_copy(lv_hbm.at[f].at[isl], l_in.at[b, hs], in_sem.at[b])
            for dt in range(DT):
                pltpu.async_copy(eps_hbm.at[f, dt, bt0 + h],
                                 e_t.at[b, pl.ds(dt * 8, 8), hs], in_sem.at[b])

    def wait_in(b):
        for h in range(H):
            hs = pl.ds(h * 128, 128)
            for dst in (m_in.at[b, hs], l_in.at[b, hs]):
                pltpu.make_async_copy(mean_hbm.at[0].at[idx_v.at[pl.ds(0, 128)]],
                                      dst, in_sem.at[b]).wait()
            for dt in range(DT):
                pltpu.make_async_copy(eps_hbm.at[0, 0, 0],
                                      e_t.at[b, pl.ds(dt * 8, 8), hs],
                                      in_sem.at[b]).wait()

    def start_out(c, b):
        f, bt0 = field_off(c)
        for h in range(H):
            hs = pl.ds(h * 128, 128)
            for dt in range(DT):
                sl = pl.ds(dt * 8, 8)
                pltpu.async_copy(m_t.at[b, sl, hs], means_out.at[f, dt, bt0 + h],
                                 out_sem.at[b])
                pltpu.async_copy(l_t.at[b, sl, hs], lvs_out.at[f, dt, bt0 + h],
                                 out_sem.at[b])
                pltpu.async_copy(z_t.at[b, sl, hs], zs_out.at[f, dt, bt0 + h],
                                 out_sem.at[b])

    def wait_out(b):
        for h in range(H):
            hs = pl.ds(h * 128, 128)
            for dt in range(DT):
                sl = pl.ds(dt * 8, 8)
                for src in (m_t.at[b, sl, hs], l_t.at[b, sl, hs], z_t.at[b, sl, hs]):
                    pltpu.make_async_copy(src, means_out.at[0, 0, 0],
                                          out_sem.at[b]).wait()

    rows_g = [lane + (g * 16) for g in range(C // 16)]

    def compute(b):
        mb, lb, eb = m_in.at[b], l_in.at[b], e_t.at[b]
        mo, lo, zo = m_t.at[b], l_t.at[b], z_t.at[b]

        @plsc.parallel_loop(0, D, unroll=2)
        def _(i):
            dcol = jnp.broadcast_to(i, (16,))
            for g in range(C // 16):
                m = plsc.load_gather(mb, [rows_g[g], dcol])
                lv = plsc.load_gather(lb, [rows_g[g], dcol])
                e = eb[i, pl.ds(g * 16, 16)]
                mo[i, pl.ds(g * 16, 16)] = m
                lo[i, pl.ds(g * 16, 16)] = lv
                zo[i, pl.ds(g * 16, 16)] = m + jnp.exp(lv * 0.5) * e

    for b in range(NBUF):
        start_in(b, b)

    def outer(g, carry):
        for b in range(NBUF):
            c = g * NBUF + b
            wait_in(b)

            @pl.when(g > 0)
            def _():
                wait_out(b)

            compute(b)
            start_out(c, b)

            @pl.when(g < NOUTER - 1)
            def _():
                start_in(c + NBUF, b)

        return carry

    lax.fori_loop(0, NOUTER, outer, 0)

    for b in range(NBUF):
        wait_out(b)


def kernel(indices, eps, mean_tables, logvar_tables):
    gidx = indices.reshape(ROWS)
    eps5 = eps.reshape(N_FIELDS, B // 128, 128, DT, 8).transpose(0, 3, 1, 4, 2)

    t5 = jax.ShapeDtypeStruct((N_FIELDS, DT, B // 128, 8, 128), jnp.float32)
    f = pl.kernel(
        _sc_body,
        out_type=(t5, t5, t5),
        mesh=plsc.VectorSubcoreMesh(core_axis_name="c", subcore_axis_name="s"),
        compiler_params=pltpu.CompilerParams(
            use_tc_tiling_on_sc=False, needs_layout_passes=False),
        scratch_types=[
            pltpu.VMEM((RPW,), jnp.int32),
            pltpu.VMEM((NBUF, C, D), jnp.float32),
            pltpu.VMEM((NBUF, C, D), jnp.float32),
            pltpu.VMEM((NBUF, D, C), jnp.float32),
            pltpu.VMEM((NBUF, D, C), jnp.float32),
            pltpu.VMEM((NBUF, D, C), jnp.float32),
            pltpu.VMEM((NBUF, D, C), jnp.float32),
            pltpu.SemaphoreType.DMA((NBUF,)),
            pltpu.SemaphoreType.DMA((NBUF,)),
        ],
    )
    m5, l5, z5 = f(gidx, eps5, mean_tables, logvar_tables)

    def back(a5):
        return a5.transpose(0, 2, 4, 1, 3).reshape(N_FIELDS, B, D)

    return back(m5), back(l5), back(z5)


# speedup vs baseline: 1.0217x; 1.0217x over previous
import jax
import jax.numpy as jnp
from jax import lax
from jax.experimental import pallas as pl
from jax.experimental.pallas import tpu as pltpu
from jax.experimental.pallas import tpu_sc as plsc

N_FIELDS = 26
VOCAB = 100000
D = 32
B = 16384

ROWS = N_FIELDS * B
NW = 32
RPW = ROWS // NW
C = 128
CPB = B // C
NCHUNK = RPW // C
NBUF = 4
NOUTER = NCHUNK // NBUF
DT = D // 8


def _sc_body(idx_hbm, eps_hbm, mean_hbm, lv_hbm,
             means_out, lvs_out, zs_out,
             idx_v, m_in, l_in, e_t, m_t, l_t, z_t,
             in_sem, out_sem):
    wid = lax.axis_index("s") * 2 + lax.axis_index("c")
    base = wid * RPW
    pltpu.sync_copy(idx_hbm.at[pl.ds(base, RPW)], idx_v)

    lane = lax.iota(jnp.int32, 16)

    def field_off(c):
        q = wid * NCHUNK + c
        f = q // CPB
        bt = q % CPB
        return f, bt

    def start_in(c, b):
        f, bt = field_off(c)
        isl = idx_v.at[pl.ds(c * C, C)]
        pltpu.async_copy(mean_hbm.at[f].at[isl], m_in.at[b], in_sem.at[b])
        pltpu.async_copy(lv_hbm.at[f].at[isl], l_in.at[b], in_sem.at[b])
        for dt in range(DT):
            pltpu.async_copy(eps_hbm.at[f, dt, bt], e_t.at[b, pl.ds(dt * 8, 8)],
                             in_sem.at[b])

    def wait_in(b):
        for dst in (m_in.at[b], l_in.at[b]):
            pltpu.make_async_copy(mean_hbm.at[0].at[idx_v.at[pl.ds(0, C)]],
                                  dst, in_sem.at[b]).wait()
        for dt in range(DT):
            pltpu.make_async_copy(eps_hbm.at[0, 0, 0], e_t.at[b, pl.ds(dt * 8, 8)],
                                  in_sem.at[b]).wait()

    def start_out(c, b):
        f, bt = field_off(c)
        for dt in range(DT):
            sl = pl.ds(dt * 8, 8)
            pltpu.async_copy(m_t.at[b, sl], means_out.at[f, dt, bt], out_sem.at[b])
            pltpu.async_copy(l_t.at[b, sl], lvs_out.at[f, dt, bt], out_sem.at[b])
            pltpu.async_copy(z_t.at[b, sl], zs_out.at[f, dt, bt], out_sem.at[b])

    def wait_out(b):
        for dt in range(DT):
            sl = pl.ds(dt * 8, 8)
            for src in (m_t.at[b, sl], l_t.at[b, sl], z_t.at[b, sl]):
                pltpu.make_async_copy(src, means_out.at[0, 0, 0], out_sem.at[b]).wait()

    rows_g = [lane + (g * 16) for g in range(8)]

    def compute(b):
        mb, lb, eb = m_in.at[b], l_in.at[b], e_t.at[b]
        mo, lo, zo = m_t.at[b], l_t.at[b], z_t.at[b]

        @plsc.parallel_loop(0, D, unroll=2)
        def _(i):
            dcol = jnp.broadcast_to(i, (16,))
            for g in range(8):
                m = plsc.load_gather(mb, [rows_g[g], dcol])
                lv = plsc.load_gather(lb, [rows_g[g], dcol])
                e = eb[i, pl.ds(g * 16, 16)]
                mo[i, pl.ds(g * 16, 16)] = m
                lo[i, pl.ds(g * 16, 16)] = lv
                zo[i, pl.ds(g * 16, 16)] = m + jnp.exp(lv * 0.5) * e

    for b in range(NBUF):
        start_in(b, b)

    def outer(g, carry):
        for b in range(NBUF):
            c = g * NBUF + b
            wait_in(b)

            @pl.when(g > 0)
            def _():
                wait_out(b)

            compute(b)
            start_out(c, b)

            @pl.when(g < NOUTER - 1)
            def _():
                start_in(c + NBUF, b)

        return carry

    lax.fori_loop(0, NOUTER, outer, 0)

    for b in range(NBUF):
        wait_out(b)


def kernel(indices, eps, mean_tables, logvar_tables):
    gidx = indices.reshape(ROWS)
    eps5 = eps.reshape(N_FIELDS, CPB, C, DT, 8).transpose(0, 3, 1, 4, 2)

    t5 = jax.ShapeDtypeStruct((N_FIELDS, DT, CPB, 8, C), jnp.float32)
    f = pl.kernel(
        _sc_body,
        out_type=(t5, t5, t5),
        mesh=plsc.VectorSubcoreMesh(core_axis_name="c", subcore_axis_name="s"),
        compiler_params=pltpu.CompilerParams(
            use_tc_tiling_on_sc=False, needs_layout_passes=False),
        scratch_types=[
            pltpu.VMEM((RPW,), jnp.int32),
            pltpu.VMEM((NBUF, C, D), jnp.float32),
            pltpu.VMEM((NBUF, C, D), jnp.float32),
            pltpu.VMEM((NBUF, D, C), jnp.float32),
            pltpu.VMEM((NBUF, D, C), jnp.float32),
            pltpu.VMEM((NBUF, D, C), jnp.float32),
            pltpu.VMEM((NBUF, D, C), jnp.float32),
            pltpu.SemaphoreType.DMA((NBUF,)),
            pltpu.SemaphoreType.DMA((NBUF,)),
        ],
    )
    m5, l5, z5 = f(gidx, eps5, mean_tables, logvar_tables)

    def back(a5):
        return a5.transpose(0, 2, 4, 1, 3).reshape(N_FIELDS, B, D)

    return back(m5), back(l5), back(z5)
